# Initial kernel scaffold; baseline (speedup 1.0000x reference)
import jax
import jax.numpy as jnp
from jax.experimental import pallas as pl


def kernel(Xi, Xp, Xv, X_seq, fo_tables, so_tables, seq_table, W1, b1, W2, b2, Wc, bc, bias):
    raise NotImplementedError("write your pallas kernel here")


# speedup vs baseline: 1.1058x; 1.1058x over previous
import functools

import jax
import jax.numpy as jnp
from jax import lax
from jax.experimental import pallas as pl
from jax.experimental.pallas import tpu as pltpu
from jax.experimental.pallas import tpu_sc as plsc

B = 16384
FIELD = 26
V = 100000
EMB = 4
HIST = 50
CH = 80000
SEQ_EMB = 64
D1 = 32
D2 = 32
FE = FIELD * EMB

NW = 32
BPW = B // NW
NB = 32
NCH = BPW // NB

BLK = 2048


def _prep_body(seq_ref, w1b_ref, t2_ref):
    t2_ref[...] = jnp.dot(
        seq_ref[...], w1b_ref[...], preferred_element_type=jnp.float32
    ) * (1.0 / HIST)


def _xif_body(xi_ref, xif_ref):
    f = lax.broadcasted_iota(jnp.int32, xi_ref.shape, 1)
    xif_ref[...] = xi_ref[...] + f * V


def _sc_body(t2, xseq, xif, sot, fot,
             seqp_o, sog_o, fog_o,
             xseq_v, rows_v, xif_v, idx4_v, sorow_v, forow_v, seqp_v):
    c = lax.axis_index("c")
    s = lax.axis_index("s")
    wid = c * 16 + s
    base0 = wid * BPW

    def chunk(i, carry):
        b0 = base0 + i * NB
        pltpu.sync_copy(xseq.at[pl.ds(b0 * HIST, NB * HIST)], xseq_v)
        pltpu.sync_copy(xif.at[pl.ds(b0 * FIELD, NB * FIELD)], xif_v)

        def expand(j, carry2):
            g = j * 16
            lane = lax.iota(jnp.int32, 16) + g
            src = lane >> 2
            val = plsc.load_gather(xif_v, [src])
            idx4_v[pl.ds(g, 16)] = val * EMB + (lane & 3)
            return carry2

        lax.fori_loop(0, NB * FIELD * EMB // 16, expand, 0)

        pltpu.sync_copy(t2.at[xseq_v], rows_v)
        pltpu.sync_copy(sot.at[idx4_v], sorow_v)
        pltpu.sync_copy(fot.at[xif_v], forow_v)

        def bb(b, carry2):
            base = b * HIST
            acc0 = jnp.zeros((16,), jnp.float32)
            acc1 = jnp.zeros((16,), jnp.float32)
            for h in range(HIST):
                acc0 = acc0 + rows_v[base + h, 0:16]
                acc1 = acc1 + rows_v[base + h, 16:32]
            seqp_v[b, 0:16] = acc0
            seqp_v[b, 16:32] = acc1
            return carry2

        lax.fori_loop(0, NB, bb, 0)
        pltpu.sync_copy(seqp_v, seqp_o.at[pl.ds(b0, NB)])
        pltpu.sync_copy(sorow_v, sog_o.at[pl.ds(b0 * FE, NB * FE)])
        pltpu.sync_copy(forow_v, fog_o.at[pl.ds(b0 * FIELD, NB * FIELD)])
        return carry

    lax.fori_loop(0, NCH, chunk, 0)


def _mlp_body(sog_ref, fog_ref, seqp_ref, xv_ref, xv4_ref,
              w1a_ref, w2_ref, wc1_ref, wc2_ref, wc3_ref,
              b1_ref, b2_ref, s_ref, out_ref):
    so = sog_ref[...] * xv4_ref[...]
    r = lax.broadcasted_iota(jnp.int32, (FE, EMB), 0)
    cc = lax.broadcasted_iota(jnp.int32, (FE, EMB), 1)
    sel = jnp.where((r % EMB) == cc, 1.0, 0.0).astype(jnp.float32)
    sum_emb = jnp.dot(so, sel, preferred_element_type=jnp.float32)
    sq_emb = jnp.dot(so * so, sel, preferred_element_type=jnp.float32)
    fm2 = 0.5 * (sum_emb * sum_emb - sq_emb)
    h1 = jnp.maximum(
        jnp.dot(so, w1a_ref[...], preferred_element_type=jnp.float32)
        + seqp_ref[...] + b1_ref[...], 0.0)
    h2 = jnp.maximum(
        jnp.dot(h1, w2_ref[...], preferred_element_type=jnp.float32)
        + b2_ref[...], 0.0)
    fm1 = fog_ref[...] * xv_ref[...]
    out = (jnp.dot(fm1, wc1_ref[...], preferred_element_type=jnp.float32)
           + jnp.dot(fm2, wc2_ref[...], preferred_element_type=jnp.float32)
           + jnp.dot(h2, wc3_ref[...], preferred_element_type=jnp.float32)
           + s_ref[0, 0])
    out_ref[...] = jax.nn.sigmoid(out)


def _make_sc_kernel():
    mesh = plsc.VectorSubcoreMesh(core_axis_name="c", subcore_axis_name="s",
                                  num_cores=2, num_subcores=16)
    return functools.partial(
        pl.kernel,
        out_type=(
            jax.ShapeDtypeStruct((B, D1), jnp.float32),
            jax.ShapeDtypeStruct((B * FE,), jnp.float32),
            jax.ShapeDtypeStruct((B * FIELD,), jnp.float32),
        ),
        mesh=mesh,
        scratch_types=[
            pltpu.VMEM((NB * HIST,), jnp.int32),
            pltpu.VMEM((NB * HIST, D1), jnp.float32),
            pltpu.VMEM((NB * FIELD,), jnp.int32),
            pltpu.VMEM((NB * FE,), jnp.int32),
            pltpu.VMEM((NB * FE,), jnp.float32),
            pltpu.VMEM((NB * FIELD,), jnp.float32),
            pltpu.VMEM((NB, D1), jnp.float32),
        ],
        compiler_params=pltpu.CompilerParams(
            use_tc_tiling_on_sc=False, needs_layout_passes=False),
    )(_sc_body)


def kernel(Xi, Xp, Xv, X_seq, fo_tables, so_tables, seq_table,
           W1, b1, W2, b2, Wc, bc, bias):
    idx = Xi[:, :, 0]
    so_flat = so_tables.reshape(FIELD * V * EMB)
    fo_flat = fo_tables.reshape(FIELD * V)
    W1a = W1[:FE]
    W1b = W1[FE:]

    RB = 4000
    t2 = pl.pallas_call(
        _prep_body,
        grid=(CH // RB,),
        in_specs=[
            pl.BlockSpec((RB, SEQ_EMB), lambda i: (i, 0)),
            pl.BlockSpec((SEQ_EMB, D1), lambda i: (0, 0)),
        ],
        out_specs=pl.BlockSpec((RB, D1), lambda i: (i, 0)),
        out_shape=jax.ShapeDtypeStruct((CH, D1), jnp.float32),
    )(seq_table, W1b)
    XB = 4096
    xif = pl.pallas_call(
        _xif_body,
        grid=(B // XB,),
        in_specs=[pl.BlockSpec((XB, FIELD), lambda i: (i, 0))],
        out_specs=pl.BlockSpec((XB, FIELD), lambda i: (i, 0)),
        out_shape=jax.ShapeDtypeStruct((B, FIELD), jnp.int32),
    )(idx)

    seqp, sog, fog = _make_sc_kernel()(
        t2, X_seq.reshape(B * HIST), xif.reshape(B * FIELD), so_flat, fo_flat)
    fog = fog.reshape(B, FIELD)

    xv4 = jnp.repeat(Xv, EMB, axis=1)
    sog2 = sog.reshape(B, FE)
    sc = (bc + bias).reshape(1, 1)
    grid = B // BLK
    out = pl.pallas_call(
        _mlp_body,
        grid=(grid,),
        in_specs=[
            pl.BlockSpec((BLK, FE), lambda i: (i, 0)),
            pl.BlockSpec((BLK, FIELD), lambda i: (i, 0)),
            pl.BlockSpec((BLK, D1), lambda i: (i, 0)),
            pl.BlockSpec((BLK, FIELD), lambda i: (i, 0)),
            pl.BlockSpec((BLK, FE), lambda i: (i, 0)),
            pl.BlockSpec((FE, D1), lambda i: (0, 0)),
            pl.BlockSpec((D1, D2), lambda i: (0, 0)),
            pl.BlockSpec((FIELD, 1), lambda i: (0, 0)),
            pl.BlockSpec((EMB, 1), lambda i: (0, 0)),
            pl.BlockSpec((D2, 1), lambda i: (0, 0)),
            pl.BlockSpec((1, D1), lambda i: (0, 0)),
            pl.BlockSpec((1, D2), lambda i: (0, 0)),
            pl.BlockSpec((1, 1), lambda i: (0, 0)),
        ],
        out_specs=pl.BlockSpec((BLK, 1), lambda i: (i, 0)),
        out_shape=jax.ShapeDtypeStruct((B, 1), jnp.float32),
    )(sog2, fog, seqp, Xv, xv4,
      W1a, W2, Wc[:FIELD], Wc[FIELD:FIELD + EMB], Wc[FIELD + EMB:],
      b1.reshape(1, D1), b2.reshape(1, D2), sc)
    return out[:, 0]
